# baseline (speedup 1.0000x reference)
import functools

import jax
import jax.numpy as jnp
from jax import lax
from jax.experimental import pallas as pl
from jax.experimental.pallas import tpu as pltpu
from jax.experimental.pallas import tpu_sc as plsc

EPS = 1e-12

_NC = 2
_NS = 16
_NW = _NC * _NS

_CHUNK = 32

_TB = 2048


def _pack_rne(a, b):
    ua = lax.bitcast_convert_type(a, jnp.int32)
    ub = lax.bitcast_convert_type(b, jnp.int32)
    one = jnp.int32(1)
    bias = jnp.int32(0x7FFF)
    ra = lax.shift_right_logical(
        ua + bias + (lax.shift_right_logical(ua, 16) & one), 16)
    rb = ub + bias + (lax.shift_right_logical(ub, 16) & one)
    return ra | (rb & jnp.int32(-65536))


def _sc_gather(table, ids):
    n_tok = ids.shape[0]
    hidden = table.shape[1]
    half = hidden // 2
    per_w = n_tok // _NW
    n_chunks = per_w // _CHUNK
    npair = half // 16

    mesh = plsc.VectorSubcoreMesh(core_axis_name="c", subcore_axis_name="s")

    @functools.partial(
        pl.kernel,
        mesh=mesh,
        out_type=jax.ShapeDtypeStruct((n_tok, half), jnp.int32),
        scratch_types=[
            pltpu.VMEM((per_w,), jnp.int32),
            pltpu.VMEM((_CHUNK, hidden), jnp.float32),
            pltpu.VMEM((_CHUNK, hidden), jnp.float32),
            pltpu.VMEM((_CHUNK, half), jnp.int32),
            pltpu.VMEM((_CHUNK, half), jnp.int32),
            pltpu.SemaphoreType.DMA,
            pltpu.SemaphoreType.DMA,
            pltpu.SemaphoreType.DMA,
            pltpu.SemaphoreType.DMA,
        ],
    )
    def gather_kernel(table_hbm, ids_hbm, out_hbm, idx_v, fb0, fb1, ob0, ob1,
                      sg0, sg1, so0, so1):
        wid = lax.axis_index("s") * _NC + lax.axis_index("c")
        base = wid * per_w
        pltpu.sync_copy(ids_hbm.at[pl.ds(base, per_w)], idx_v)
        fbs = (fb0, fb1)
        obs = (ob0, ob1)
        sgs = (sg0, sg1)
        sos = (so0, so1)

        def fire_gather(c, b):
            pltpu.async_copy(
                table_hbm.at[idx_v.at[pl.ds(c * _CHUNK, _CHUNK)]], fbs[b], sgs[b])

        def wait_gather(b):
            pltpu.make_async_copy(
                table_hbm.at[pl.ds(0, _CHUNK)], fbs[b], sgs[b]).wait()

        def wait_out(b):
            pltpu.make_async_copy(
                out_hbm.at[pl.ds(0, _CHUNK)], obs[b], sos[b]).wait()

        fire_gather(0, 0)
        fire_gather(1, 1)

        def chunk(c, b):
            wait_gather(b)

            @pl.when(c >= 2)
            def _():
                wait_out(b)

            fb = fbs[b]
            ob = obs[b]

            @plsc.parallel_loop(0, _CHUNK, unroll=2)
            def tok(i):
                for p in range(npair):
                    a = fb[i, pl.ds(16 * p, 16)]
                    bb = fb[i, pl.ds(half + 16 * p, 16)]
                    ob[i, pl.ds(16 * p, 16)] = _pack_rne(a, bb)

            pltpu.async_copy(ob, out_hbm.at[pl.ds(base + c * _CHUNK, _CHUNK)], sos[b])

            @pl.when(c + 2 < n_chunks)
            def _():
                fire_gather(c + 2, b)

        def pair(t, carry):
            chunk(2 * t, 0)
            chunk(2 * t + 1, 1)
            return carry

        lax.fori_loop(0, n_chunks // 2, pair, 0)
        wait_out(0)
        wait_out(1)

    return gather_kernel(table, ids)


def _ln_body(g_ref, tt_ref, pos_ref, tte_ref, w_ref, b_ref, o_ref):
    gi = g_ref[...]
    lo = lax.bitcast_convert_type(lax.shift_left(gi, 16), jnp.float32)
    hi = lax.bitcast_convert_type(gi & jnp.int32(-65536), jnp.float32)
    g = jnp.concatenate([lo, hi], axis=1)
    x = g + pos_ref[...]
    ttf = tt_ref[0, 0, :].astype(jnp.float32)
    t0 = tte_ref[0, :]
    t1 = tte_ref[1, :]
    x = x + t0[None, :] + ttf[:, None] * (t1 - t0)[None, :]
    u = jnp.mean(x, axis=-1, keepdims=True)
    s = jnp.mean((x - u) ** 2, axis=-1, keepdims=True)
    y = (x - u) * lax.rsqrt(s + EPS)
    o_ref[...] = y * w_ref[0, :][None, :] + b_ref[0, :][None, :]


def _tc_add_ln_slice(buf, gathered, tt_ids, pos_emb, tt_emb, ln_w, ln_b,
                     n_tok, seq, blk0):
    slice_tok, half = gathered.shape
    hidden = pos_emb.shape[1]
    rows = slice_tok // _TB
    sb = seq // _TB

    tt3 = tt_ids.reshape(rows * sb, 1, _TB)
    args = [gathered, tt3, pos_emb, tt_emb,
            ln_w.reshape(1, hidden), ln_b.reshape(1, hidden)]
    in_specs = [
        pl.BlockSpec((_TB, half), lambda i, b: (b * sb + i, 0)),
        pl.BlockSpec((1, 1, _TB), lambda i, b: (b * sb + i, 0, 0)),
        pl.BlockSpec((_TB, hidden), lambda i, b: (i, 0)),
        pl.BlockSpec((2, hidden), lambda i, b: (0, 0)),
        pl.BlockSpec((1, hidden), lambda i, b: (0, 0)),
        pl.BlockSpec((1, hidden), lambda i, b: (0, 0)),
    ]
    body = _ln_body
    aliases = {}
    if buf is not None:
        args = [buf] + args
        in_specs = [pl.BlockSpec(memory_space=pl.ANY)] + in_specs
        aliases = {0: 0}

        def body(buf_ref, *refs):
            _ln_body(*refs)

    return pl.pallas_call(
        body,
        grid=(sb, rows),
        in_specs=in_specs,
        out_specs=pl.BlockSpec((_TB, hidden), lambda i, b: (blk0 + b * sb + i, 0)),
        out_shape=jax.ShapeDtypeStruct((n_tok, hidden), jnp.float32),
        input_output_aliases=aliases,
    )(*args)


def kernel(input_ids, token_type_ids, word_emb, token_type_emb, pos_emb, ln_weight, ln_bias):
    batch, seq = input_ids.shape
    hidden = word_emb.shape[1]
    n_tok = batch * seq
    ids = input_ids.reshape(-1).astype(jnp.int32)
    tt_ids = token_type_ids.reshape(-1).astype(jnp.int32)

    half = n_tok // 2
    g0 = _sc_gather(word_emb, ids[:half])
    g1 = _sc_gather(word_emb, ids[half:])

    out = _tc_add_ln_slice(None, g0, tt_ids[:half], pos_emb, token_type_emb,
                           ln_weight, ln_bias, n_tok, seq, 0)
    out = _tc_add_ln_slice(out, g1, tt_ids[half:], pos_emb, token_type_emb,
                           ln_weight, ln_bias, n_tok, seq, half // _TB)
    return out.reshape(batch, seq, hidden)


# speedup vs baseline: 1.2644x; 1.2644x over previous
import functools

import jax
import jax.numpy as jnp
from jax import lax
from jax.experimental import pallas as pl
from jax.experimental.pallas import tpu as pltpu
from jax.experimental.pallas import tpu_sc as plsc

EPS = 1e-12

_NC = 2
_NS = 16
_NW = _NC * _NS

_CHUNK = 64

_TB = 2048


def _sc_gather(table, ids):
    n_tok = ids.shape[0]
    hidden = table.shape[1]
    per_w = n_tok // _NW
    n_chunks = per_w // _CHUNK

    mesh = plsc.VectorSubcoreMesh(core_axis_name="c", subcore_axis_name="s")

    @functools.partial(
        pl.kernel,
        mesh=mesh,
        out_type=jax.ShapeDtypeStruct((n_tok, hidden), jnp.float32),
        scratch_types=[
            pltpu.VMEM((per_w,), jnp.int32),
            pltpu.VMEM((_CHUNK, hidden), jnp.float32),
            pltpu.VMEM((_CHUNK, hidden), jnp.float32),
            pltpu.SemaphoreType.DMA,
            pltpu.SemaphoreType.DMA,
        ],
    )
    def gather_kernel(table_hbm, ids_hbm, out_hbm, idx_v, buf0, buf1, sem0, sem1):
        wid = lax.axis_index("s") * _NC + lax.axis_index("c")
        base = wid * per_w
        pltpu.sync_copy(ids_hbm.at[pl.ds(base, per_w)], idx_v)
        bufs = (buf0, buf1)
        sems = (sem0, sem1)
        copies = [None] * n_chunks
        copies[0] = pltpu.async_copy(
            table_hbm.at[idx_v.at[pl.ds(0, _CHUNK)]], buf0, sem0
        )
        for k in range(n_chunks):
            nxt = k + 1
            if nxt < n_chunks:
                copies[nxt] = pltpu.async_copy(
                    table_hbm.at[idx_v.at[pl.ds(nxt * _CHUNK, _CHUNK)]],
                    bufs[nxt % 2],
                    sems[nxt % 2],
                )
            copies[k].wait()
            pltpu.sync_copy(bufs[k % 2], out_hbm.at[pl.ds(base + k * _CHUNK, _CHUNK)])

    return gather_kernel(table, ids)


def _ln_body(g_ref, tt_ref, pos_ref, tte_ref, w_ref, b_ref, o_ref):
    x = g_ref[...] + pos_ref[...]
    ttf = tt_ref[0, 0, :].astype(jnp.float32)
    t0 = tte_ref[0, :]
    t1 = tte_ref[1, :]
    x = x + t0[None, :] + ttf[:, None] * (t1 - t0)[None, :]
    u = jnp.mean(x, axis=-1, keepdims=True)
    s = jnp.mean((x - u) ** 2, axis=-1, keepdims=True)
    y = (x - u) * lax.rsqrt(s + EPS)
    o_ref[...] = y * w_ref[0, :][None, :] + b_ref[0, :][None, :]


def _tc_add_ln_slice(buf, gathered, tt_ids, pos_emb, tt_emb, ln_w, ln_b,
                     n_tok, seq, blk0):
    slice_tok, hidden = gathered.shape
    rows = slice_tok // _TB
    sb = seq // _TB

    tt3 = tt_ids.reshape(rows * sb, 1, _TB)
    args = [gathered, tt3, pos_emb, tt_emb,
            ln_w.reshape(1, hidden), ln_b.reshape(1, hidden)]
    in_specs = [
        pl.BlockSpec((_TB, hidden), lambda i, b: (b * sb + i, 0)),
        pl.BlockSpec((1, 1, _TB), lambda i, b: (b * sb + i, 0, 0)),
        pl.BlockSpec((_TB, hidden), lambda i, b: (i, 0)),
        pl.BlockSpec((2, hidden), lambda i, b: (0, 0)),
        pl.BlockSpec((1, hidden), lambda i, b: (0, 0)),
        pl.BlockSpec((1, hidden), lambda i, b: (0, 0)),
    ]
    body = _ln_body
    aliases = {}
    if buf is not None:
        args = [buf] + args
        in_specs = [pl.BlockSpec(memory_space=pl.ANY)] + in_specs
        aliases = {0: 0}

        def body(buf_ref, *refs):
            _ln_body(*refs)

    return pl.pallas_call(
        body,
        grid=(sb, rows),
        in_specs=in_specs,
        out_specs=pl.BlockSpec((_TB, hidden), lambda i, b: (blk0 + b * sb + i, 0)),
        out_shape=jax.ShapeDtypeStruct((n_tok, hidden), jnp.float32),
        input_output_aliases=aliases,
    )(*args)


def kernel(input_ids, token_type_ids, word_emb, token_type_emb, pos_emb, ln_weight, ln_bias):
    batch, seq = input_ids.shape
    hidden = word_emb.shape[1]
    n_tok = batch * seq
    ids = input_ids.reshape(-1).astype(jnp.int32)
    tt_ids = token_type_ids.reshape(-1).astype(jnp.int32)

    half = n_tok // 2
    g0 = _sc_gather(word_emb, ids[:half])
    g1 = _sc_gather(word_emb, ids[half:])

    out = _tc_add_ln_slice(None, g0, tt_ids[:half], pos_emb, token_type_emb,
                           ln_weight, ln_bias, n_tok, seq, 0)
    out = _tc_add_ln_slice(out, g1, tt_ids[half:], pos_emb, token_type_emb,
                           ln_weight, ln_bias, n_tok, seq, half // _TB)
    return out.reshape(batch, seq, hidden)
